# baseline (speedup 1.0000x reference)
import functools
import math

import jax
import jax.numpy as jnp
from jax import lax
from jax.experimental import pallas as pl
from jax.experimental.pallas import tpu as pltpu
from jax.experimental.pallas import tpu_sc as plsc

EMB = 64
SCALE = math.sqrt(EMB)

NC = 2
NS = 16
NW = NC * NS
LANES = 16

CA = 104
NBUF = 4


def _make_gather(R, T):
    assert R % NW == 0
    rows_w = R // NW
    assert rows_w % NBUF == 0
    CB = T - CA
    assert CA % 8 == 0 and CB % 8 == 0 and CA <= 128 and CB <= 128
    mesh = plsc.VectorSubcoreMesh(
        core_axis_name="c", subcore_axis_name="s", num_cores=NC, num_subcores=NS
    )

    @functools.partial(
        pl.kernel,
        out_type=jax.ShapeDtypeStruct((R, T, EMB), jnp.float32),
        mesh=mesh,
        compiler_params=pltpu.CompilerParams(use_tc_tiling_on_sc=False),
        scratch_types=[
            pltpu.VMEM((rows_w, CA), jnp.int32),
            pltpu.VMEM((rows_w, CB), jnp.int32),
            pltpu.VMEM((NBUF, T, EMB), jnp.float32),
            pltpu.VMEM((NBUF, T, EMB), jnp.float32),
            pltpu.SemaphoreType.DMA((NBUF,)),
            pltpu.SemaphoreType.DMA((NBUF,)),
        ],
    )
    def gather_kernel(tok_hbm, table_hbm, out_hbm, idx_a, idx_b, gbuf, sbuf,
                      gsem, ssem):
        wid = lax.axis_index("s") * NC + lax.axis_index("c")
        base = wid * rows_w
        pltpu.sync_copy(tok_hbm.at[pl.ds(base, rows_w), pl.ds(0, CA)], idx_a)
        pltpu.sync_copy(tok_hbm.at[pl.ds(base, rows_w), pl.ds(CA, CB)], idx_b)

        def gather_pair(r, b):
            return (
                pltpu.make_async_copy(
                    table_hbm.at[idx_a.at[r]], gbuf.at[b].at[pl.ds(0, CA)],
                    gsem.at[b]),
                pltpu.make_async_copy(
                    table_hbm.at[idx_b.at[r]], gbuf.at[b].at[pl.ds(CA, CB)],
                    gsem.at[b]),
            )

        def store_copy(r, b):
            return pltpu.make_async_copy(sbuf.at[b], out_hbm.at[base + r],
                                         ssem.at[b])

        for b in range(NBUF):
            for c in gather_pair(b, b):
                c.start()

        @pl.loop(0, rows_w, step=NBUF)
        def _group(g):
            for b in range(NBUF):
                r = g + b
                for c in gather_pair(r, b):
                    c.wait()

                @pl.when(r >= NBUF)
                def _drain():
                    store_copy(r - NBUF, b).wait()

                src = gbuf.at[b]
                dst = sbuf.at[b]

                @plsc.parallel_loop(0, T, unroll=4)
                def _scale(t):
                    for c in range(EMB // LANES):
                        sl = pl.ds(c * LANES, LANES)
                        dst[t, sl] = src[t, sl] * SCALE

                nr = r + NBUF

                @pl.when(nr < rows_w)
                def _prefetch():
                    for c in gather_pair(nr, b):
                        c.start()

                store_copy(r, b).start()

        for b in range(NBUF):
            store_copy(rows_w - NBUF + b, b).wait()

    return gather_kernel


def kernel(tokens, table):
    R, T = tokens.shape
    out = _make_gather(R, T)(tokens.astype(jnp.int32), table)
    return out
